# baseline (speedup 1.0000x reference)
import functools

import jax
import jax.numpy as jnp
from jax import lax
from jax.experimental import pallas as pl
from jax.experimental.pallas import tpu as pltpu
from jax.experimental.pallas import tpu_sc as plsc

G, S, H = 2, 2048, 768
E, K = 64, 1
CAP = 128
T = G * S
NW = 32
CH = T // NW
NVR = CH // 16
BUF_ROWS = E * CAP + CH
TRASH = E * CAP
EPB = 2


def _wid():
    return lax.axis_index("s") * 2 + lax.axis_index("c")


def _lookup64(tab4, v):
    sel = lax.shift_right_logical(v, 4)
    off = jnp.bitwise_and(v, 15)
    r = jnp.zeros((16,), jnp.int32)
    for k in range(4):
        g = jnp.take_along_axis(tab4[k], off, axis=0)
        r = jnp.where(sel == k, g, r)
    return r


def _hist_body(idx_hbm, hists_hbm, idxv, histv):
    w = _wid()
    pltpu.sync_copy(idx_hbm.at[pl.ds(w * CH, CH)], idxv)
    lane = lax.iota(jnp.int32, 16)
    vregs = [idxv[pl.ds(16 * i, 16)] for i in range(NVR)]
    h = [jnp.zeros((16,), jnp.int32) for _ in range(4)]
    for e in range(E):
        tot = jnp.zeros((16,), jnp.int32)
        for i in range(NVR):
            tot = tot + plsc.all_reduce_population_count(vregs[i] == e)
        k = e >> 4
        h[k] = h[k] + jnp.where(lane == (e & 15), tot, 0)
    for k in range(4):
        histv[pl.ds(16 * k, 16)] = h[k]
    pltpu.sync_copy(histv, hists_hbm.at[pl.ds(w * E, E)])


def _route_body(idx_hbm, w_hbm, x_hbm, hists_hbm,
                rft_hbm, wp_hbm, buf_hbm,
                idxv, wv, histsv, rowsv, scatv, wpv, xv, sem):
    w = _wid()
    pltpu.sync_copy(idx_hbm.at[pl.ds(w * CH, CH)], idxv)
    pltpu.sync_copy(w_hbm.at[pl.ds(w * CH, CH)], wv)
    pltpu.sync_copy(hists_hbm, histsv)
    pltpu.sync_copy(x_hbm.at[pl.ds(w * CH, CH)], xv)
    lane = lax.iota(jnp.int32, 16)
    base = [jnp.zeros((16,), jnp.int32) for _ in range(4)]
    cnt = [jnp.zeros((16,), jnp.int32) for _ in range(4)]
    for wp_ in range(NW):
        pred = jnp.full((16,), wp_, jnp.int32) < w
        for k in range(4):
            hk = histsv[pl.ds(wp_ * E + 16 * k, 16)]
            cnt[k] = cnt[k] + hk
            base[k] = base[k] + jnp.where(pred, hk, 0)
    vregs = [idxv[pl.ds(16 * i, 16)] for i in range(NVR)]
    for i in range(NVR):
        v = vregs[i]
        rank = jnp.zeros((16,), jnp.int32)
        for j in range(i):
            u = vregs[j]
            for l in range(16):
                ul = jnp.take_along_axis(u, jnp.full((16,), l, jnp.int32), axis=0)
                rank = rank + (v == ul).astype(jnp.int32)
        for d in range(1, 16):
            rolled = jnp.take_along_axis(v, jnp.maximum(lane - d, 0), axis=0)
            rank = rank + ((rolled == v) & (lane >= d)).astype(jnp.int32)
        slot = _lookup64(base, v) + rank
        c_e = _lookup64(cnt, v)
        validv = slot < CAP
        row = v * CAP + jnp.minimum(slot, CAP - 1)
        scat = jnp.where(validv, row, TRASH + lane)
        dead = (slot == CAP - 1) & (c_e > CAP)
        wp = wv[pl.ds(16 * i, 16)] * (validv & ~dead).astype(jnp.float32)
        rowsv[pl.ds(16 * i, 16)] = row
        scatv[pl.ds(16 * i, 16)] = scat
        wpv[pl.ds(16 * i, 16)] = wp
    pltpu.sync_copy(rowsv, rft_hbm.at[pl.ds(w * CH, CH)])
    pltpu.sync_copy(wpv, wp_hbm.at[pl.ds(w * CH, CH)])
    pltpu.async_copy(xv, buf_hbm.at[scatv], sem).wait()


NSUB = 4


def _gather_body(rft_hbm, eo_hbm, routed_hbm, idxs, rows, semg, sems):
    w = _wid()
    sub = CH // NSUB
    for c in range(NSUB):
        pltpu.sync_copy(rft_hbm.at[pl.ds(w * CH + c * sub, sub)], idxs[c])
    gets = [pltpu.async_copy(eo_hbm.at[idxs[c]], rows[c], semg)
            for c in range(NSUB)]
    puts = []
    for c in range(NSUB):
        gets[c].wait()
        puts.append(pltpu.async_copy(
            rows[c], routed_hbm.at[pl.ds(w * CH + c * sub, sub)], sems))
    for p in puts:
        p.wait()


def _dotT(a, b):
    return lax.dot_general(a, b, (((1,), (1,)), ((), ())),
                           preferred_element_type=jnp.float32)


def _expert_ffn_body(buf_ref, gw_ref, uw_ref, dw_ref, out_ref):
    for e2 in range(EPB):
        x = buf_ref[pl.ds(e2 * CAP, CAP), :].astype(jnp.bfloat16)
        g = _dotT(x, gw_ref[e2].astype(jnp.bfloat16))
        u = _dotT(x, uw_ref[e2].astype(jnp.bfloat16))
        act = (g * jax.nn.sigmoid(g)) * u
        out_ref[pl.ds(e2 * CAP, CAP), :] = _dotT(
            act.astype(jnp.bfloat16), dw_ref[e2].astype(jnp.bfloat16))


def _shared_ffn_body(x_ref, gw_ref, uw_ref, dw_ref, out_ref):
    x = x_ref[...].astype(jnp.bfloat16)
    g = _dotT(x, gw_ref[...].astype(jnp.bfloat16))
    u = _dotT(x, uw_ref[...].astype(jnp.bfloat16))
    act = (g * jax.nn.sigmoid(g)) * u
    out_ref[...] = _dotT(act.astype(jnp.bfloat16), dw_ref[...].astype(jnp.bfloat16))


def _combine_body(sh_ref, r_ref, wp_ref, out_ref):
    wcol = jnp.reshape(wp_ref[...], (sh_ref.shape[0], 1))
    out_ref[...] = sh_ref[...] + r_ref[...] * wcol


def kernel(x, topk_indices, topk_weights, gate_w, up_w, down_w,
           shared_gate_w, shared_up_w, shared_down_w):
    xf = x.reshape(T, H)
    idx = topk_indices.reshape(T).astype(jnp.int32)
    wts = topk_weights.reshape(T)

    mesh = plsc.VectorSubcoreMesh(core_axis_name="c", subcore_axis_name="s")

    hist_k = pl.kernel(
        _hist_body,
        out_type=jax.ShapeDtypeStruct((NW * E,), jnp.int32),
        mesh=mesh,
        compiler_params=pltpu.CompilerParams(needs_layout_passes=False),
        scratch_types=[pltpu.VMEM((CH,), jnp.int32),
                       pltpu.VMEM((E,), jnp.int32)],
    )
    hists = hist_k(idx)

    route_k = pl.kernel(
        _route_body,
        out_type=[jax.ShapeDtypeStruct((T,), jnp.int32),
                  jax.ShapeDtypeStruct((T,), jnp.float32),
                  jax.ShapeDtypeStruct((BUF_ROWS, H), jnp.float32)],
        mesh=mesh,
        compiler_params=pltpu.CompilerParams(needs_layout_passes=False),
        scratch_types=[pltpu.VMEM((CH,), jnp.int32),
                       pltpu.VMEM((CH,), jnp.float32),
                       pltpu.VMEM((NW * E,), jnp.int32),
                       pltpu.VMEM((CH,), jnp.int32),
                       pltpu.VMEM((CH,), jnp.int32),
                       pltpu.VMEM((CH,), jnp.float32),
                       pltpu.VMEM((CH, H), jnp.float32),
                       pltpu.SemaphoreType.DMA],
    )
    rft, wprime, buf = route_k(idx, wts, xf, hists)

    ST = 2048
    shared = pl.pallas_call(
        _shared_ffn_body,
        grid=(T // ST,),
        in_specs=[
            pl.BlockSpec((ST, H), lambda i: (i, 0)),
            pl.BlockSpec((H, H), lambda i: (0, 0)),
            pl.BlockSpec((H, H), lambda i: (0, 0)),
            pl.BlockSpec((H, H), lambda i: (0, 0)),
        ],
        out_specs=pl.BlockSpec((ST, H), lambda i: (i, 0)),
        out_shape=jax.ShapeDtypeStruct((T, H), jnp.float32),
    )(xf, shared_gate_w, shared_up_w, shared_down_w)

    shared, buf = lax.optimization_barrier((shared, buf))

    expert_out = pl.pallas_call(
        _expert_ffn_body,
        grid=(E // EPB,),
        in_specs=[
            pl.BlockSpec((EPB * CAP, H), lambda e: (e, 0)),
            pl.BlockSpec((EPB, H, H), lambda e: (e, 0, 0)),
            pl.BlockSpec((EPB, H, H), lambda e: (e, 0, 0)),
            pl.BlockSpec((EPB, H, H), lambda e: (e, 0, 0)),
        ],
        out_specs=pl.BlockSpec((EPB * CAP, H), lambda e: (e, 0)),
        out_shape=jax.ShapeDtypeStruct((E * CAP, H), jnp.float32),
        compiler_params=pltpu.CompilerParams(
            vmem_limit_bytes=100 * 1024 * 1024),
    )(buf, gate_w, up_w, down_w)

    gather_k = pl.kernel(
        _gather_body,
        out_type=jax.ShapeDtypeStruct((T, H), jnp.float32),
        mesh=mesh,
        compiler_params=pltpu.CompilerParams(needs_layout_passes=False),
        scratch_types=[[pltpu.VMEM((CH // NSUB,), jnp.int32)] * NSUB,
                       [pltpu.VMEM((CH // NSUB, H), jnp.float32)] * NSUB,
                       pltpu.SemaphoreType.DMA,
                       pltpu.SemaphoreType.DMA],
    )
    routed = gather_k(rft, expert_out)

    TT = 512
    out = pl.pallas_call(
        _combine_body,
        grid=(T // TT,),
        in_specs=[
            pl.BlockSpec((TT, H), lambda i: (i, 0)),
            pl.BlockSpec((TT, H), lambda i: (i, 0)),
            pl.BlockSpec((TT,), lambda i: (i,)),
        ],
        out_specs=pl.BlockSpec((TT, H), lambda i: (i, 0)),
        out_shape=jax.ShapeDtypeStruct((T, H), jnp.float32),
    )(shared, routed, wprime)

    return out.reshape(G, S, H)


# speedup vs baseline: 1.0052x; 1.0052x over previous
import functools

import jax
import jax.numpy as jnp
from jax import lax
from jax.experimental import pallas as pl
from jax.experimental.pallas import tpu as pltpu
from jax.experimental.pallas import tpu_sc as plsc

G, S, H = 2, 2048, 768
E, K = 64, 1
CAP = 128
T = G * S
NW = 32
CH = T // NW
NVR = CH // 16
BUF_ROWS = E * CAP + CH
TRASH = E * CAP
EPB = 2


def _wid():
    return lax.axis_index("s") * 2 + lax.axis_index("c")


def _lookup64(tab4, v):
    sel = lax.shift_right_logical(v, 4)
    off = jnp.bitwise_and(v, 15)
    r = jnp.zeros((16,), jnp.int32)
    for k in range(4):
        g = jnp.take_along_axis(tab4[k], off, axis=0)
        r = jnp.where(sel == k, g, r)
    return r


def _hist_body(idx_hbm, hists_hbm, idxv, histv):
    w = _wid()
    pltpu.sync_copy(idx_hbm.at[pl.ds(w * CH, CH)], idxv)
    lane = lax.iota(jnp.int32, 16)
    vregs = [idxv[pl.ds(16 * i, 16)] for i in range(NVR)]
    h = [jnp.zeros((16,), jnp.int32) for _ in range(4)]
    for e in range(E):
        tot = jnp.zeros((16,), jnp.int32)
        for i in range(NVR):
            tot = tot + plsc.all_reduce_population_count(vregs[i] == e)
        k = e >> 4
        h[k] = h[k] + jnp.where(lane == (e & 15), tot, 0)
    for k in range(4):
        histv[pl.ds(16 * k, 16)] = h[k]
    pltpu.sync_copy(histv, hists_hbm.at[pl.ds(w * E, E)])


def _route_body(idx_hbm, w_hbm, x_hbm, hists_hbm,
                rft_hbm, wp_hbm, buf_hbm,
                idxv, wv, histsv, rowsv, scatv, wpv, xv, sem):
    w = _wid()
    xload = pltpu.async_copy(x_hbm.at[pl.ds(w * CH, CH)], xv, sem)
    pltpu.sync_copy(idx_hbm.at[pl.ds(w * CH, CH)], idxv)
    pltpu.sync_copy(w_hbm.at[pl.ds(w * CH, CH)], wv)
    pltpu.sync_copy(hists_hbm, histsv)
    lane = lax.iota(jnp.int32, 16)
    base = [jnp.zeros((16,), jnp.int32) for _ in range(4)]
    cnt = [jnp.zeros((16,), jnp.int32) for _ in range(4)]
    for wp_ in range(NW):
        pred = jnp.full((16,), wp_, jnp.int32) < w
        for k in range(4):
            hk = histsv[pl.ds(wp_ * E + 16 * k, 16)]
            cnt[k] = cnt[k] + hk
            base[k] = base[k] + jnp.where(pred, hk, 0)
    vregs = [idxv[pl.ds(16 * i, 16)] for i in range(NVR)]
    for i in range(NVR):
        v = vregs[i]
        rank = jnp.zeros((16,), jnp.int32)
        for j in range(i):
            u = vregs[j]
            for l in range(16):
                ul = jnp.take_along_axis(u, jnp.full((16,), l, jnp.int32), axis=0)
                rank = rank + (v == ul).astype(jnp.int32)
        for d in range(1, 16):
            rolled = jnp.take_along_axis(v, jnp.maximum(lane - d, 0), axis=0)
            rank = rank + ((rolled == v) & (lane >= d)).astype(jnp.int32)
        slot = _lookup64(base, v) + rank
        c_e = _lookup64(cnt, v)
        validv = slot < CAP
        row = v * CAP + jnp.minimum(slot, CAP - 1)
        scat = jnp.where(validv, row, TRASH + lane)
        dead = (slot == CAP - 1) & (c_e > CAP)
        wp = wv[pl.ds(16 * i, 16)] * (validv & ~dead).astype(jnp.float32)
        rowsv[pl.ds(16 * i, 16)] = row
        scatv[pl.ds(16 * i, 16)] = scat
        wpv[pl.ds(16 * i, 16)] = wp
    pltpu.sync_copy(rowsv, rft_hbm.at[pl.ds(w * CH, CH)])
    pltpu.sync_copy(wpv, wp_hbm.at[pl.ds(w * CH, CH)])
    xload.wait()
    pltpu.async_copy(xv, buf_hbm.at[scatv], sem).wait()


def _gather_body(rft_hbm, eo_hbm, routed_hbm, idxv, rowsb, sem):
    w = _wid()
    pltpu.sync_copy(rft_hbm.at[pl.ds(w * CH, CH)], idxv)
    pltpu.async_copy(eo_hbm.at[idxv], rowsb, sem).wait()
    pltpu.sync_copy(rowsb, routed_hbm.at[pl.ds(w * CH, CH)])


def _dotT(a, b):
    return lax.dot_general(a, b, (((1,), (1,)), ((), ())),
                           preferred_element_type=jnp.float32)


def _expert_ffn_body(buf_ref, gw_ref, uw_ref, dw_ref, out_ref):
    for e2 in range(EPB):
        x = buf_ref[pl.ds(e2 * CAP, CAP), :].astype(jnp.bfloat16)
        g = _dotT(x, gw_ref[e2].astype(jnp.bfloat16))
        u = _dotT(x, uw_ref[e2].astype(jnp.bfloat16))
        act = (g * jax.nn.sigmoid(g)) * u
        out_ref[pl.ds(e2 * CAP, CAP), :] = _dotT(
            act.astype(jnp.bfloat16), dw_ref[e2].astype(jnp.bfloat16))


def _shared_ffn_body(x_ref, gw_ref, uw_ref, dw_ref, out_ref):
    x = x_ref[...].astype(jnp.bfloat16)
    g = _dotT(x, gw_ref[...].astype(jnp.bfloat16))
    u = _dotT(x, uw_ref[...].astype(jnp.bfloat16))
    act = (g * jax.nn.sigmoid(g)) * u
    out_ref[...] = _dotT(act.astype(jnp.bfloat16), dw_ref[...].astype(jnp.bfloat16))


def _combine_body(sh_ref, r_ref, wp_ref, out_ref):
    wcol = jnp.reshape(wp_ref[...], (sh_ref.shape[0], 1))
    out_ref[...] = sh_ref[...] + r_ref[...] * wcol


def kernel(x, topk_indices, topk_weights, gate_w, up_w, down_w,
           shared_gate_w, shared_up_w, shared_down_w):
    xf = x.reshape(T, H)
    idx = topk_indices.reshape(T).astype(jnp.int32)
    wts = topk_weights.reshape(T)

    mesh = plsc.VectorSubcoreMesh(core_axis_name="c", subcore_axis_name="s")

    hist_k = pl.kernel(
        _hist_body,
        out_type=jax.ShapeDtypeStruct((NW * E,), jnp.int32),
        mesh=mesh,
        compiler_params=pltpu.CompilerParams(needs_layout_passes=False),
        scratch_types=[pltpu.VMEM((CH,), jnp.int32),
                       pltpu.VMEM((E,), jnp.int32)],
    )
    hists = hist_k(idx)

    route_k = pl.kernel(
        _route_body,
        out_type=[jax.ShapeDtypeStruct((T,), jnp.int32),
                  jax.ShapeDtypeStruct((T,), jnp.float32),
                  jax.ShapeDtypeStruct((BUF_ROWS, H), jnp.float32)],
        mesh=mesh,
        compiler_params=pltpu.CompilerParams(needs_layout_passes=False),
        scratch_types=[pltpu.VMEM((CH,), jnp.int32),
                       pltpu.VMEM((CH,), jnp.float32),
                       pltpu.VMEM((NW * E,), jnp.int32),
                       pltpu.VMEM((CH,), jnp.int32),
                       pltpu.VMEM((CH,), jnp.int32),
                       pltpu.VMEM((CH,), jnp.float32),
                       pltpu.VMEM((CH, H), jnp.float32),
                       pltpu.SemaphoreType.DMA],
    )
    rft, wprime, buf = route_k(idx, wts, xf, hists)

    ST = 2048
    shared = pl.pallas_call(
        _shared_ffn_body,
        grid=(T // ST,),
        in_specs=[
            pl.BlockSpec((ST, H), lambda i: (i, 0)),
            pl.BlockSpec((H, H), lambda i: (0, 0)),
            pl.BlockSpec((H, H), lambda i: (0, 0)),
            pl.BlockSpec((H, H), lambda i: (0, 0)),
        ],
        out_specs=pl.BlockSpec((ST, H), lambda i: (i, 0)),
        out_shape=jax.ShapeDtypeStruct((T, H), jnp.float32),
    )(xf, shared_gate_w, shared_up_w, shared_down_w)

    shared, buf = lax.optimization_barrier((shared, buf))

    expert_out = pl.pallas_call(
        _expert_ffn_body,
        grid=(E // EPB,),
        in_specs=[
            pl.BlockSpec((EPB * CAP, H), lambda e: (e, 0)),
            pl.BlockSpec((EPB, H, H), lambda e: (e, 0, 0)),
            pl.BlockSpec((EPB, H, H), lambda e: (e, 0, 0)),
            pl.BlockSpec((EPB, H, H), lambda e: (e, 0, 0)),
        ],
        out_specs=pl.BlockSpec((EPB * CAP, H), lambda e: (e, 0)),
        out_shape=jax.ShapeDtypeStruct((E * CAP, H), jnp.float32),
        compiler_params=pltpu.CompilerParams(
            vmem_limit_bytes=100 * 1024 * 1024),
    )(buf, gate_w, up_w, down_w)

    gather_k = pl.kernel(
        _gather_body,
        out_type=jax.ShapeDtypeStruct((T, H), jnp.float32),
        mesh=mesh,
        compiler_params=pltpu.CompilerParams(needs_layout_passes=False),
        scratch_types=[pltpu.VMEM((CH,), jnp.int32),
                       pltpu.VMEM((CH, H), jnp.float32),
                       pltpu.SemaphoreType.DMA],
    )
    routed = gather_k(rft, expert_out)

    TT = 512
    out = pl.pallas_call(
        _combine_body,
        grid=(T // TT,),
        in_specs=[
            pl.BlockSpec((TT, H), lambda i: (i, 0)),
            pl.BlockSpec((TT, H), lambda i: (i, 0)),
            pl.BlockSpec((TT,), lambda i: (i,)),
        ],
        out_specs=pl.BlockSpec((TT, H), lambda i: (i, 0)),
        out_shape=jax.ShapeDtypeStruct((T, H), jnp.float32),
    )(shared, routed, wprime)

    return out.reshape(G, S, H)


# speedup vs baseline: 1.0250x; 1.0197x over previous
import functools

import jax
import jax.numpy as jnp
from jax import lax
from jax.experimental import pallas as pl
from jax.experimental.pallas import tpu as pltpu
from jax.experimental.pallas import tpu_sc as plsc

G, S, H = 2, 2048, 768
E, K = 64, 1
CAP = 128
T = G * S
NW = 32
CH = T // NW
NVR = CH // 16
BUF_ROWS = E * CAP + CH
TRASH = E * CAP
EPB = 2


def _wid():
    return lax.axis_index("s") * 2 + lax.axis_index("c")


def _lookup64(tab4, v):
    sel = lax.shift_right_logical(v, 4)
    off = jnp.bitwise_and(v, 15)
    r = jnp.zeros((16,), jnp.int32)
    for k in range(4):
        g = jnp.take_along_axis(tab4[k], off, axis=0)
        r = jnp.where(sel == k, g, r)
    return r


def _hist_body(idx_hbm, hists_hbm, idxv, histv):
    w = _wid()
    pltpu.sync_copy(idx_hbm.at[pl.ds(w * CH, CH)], idxv)
    lane = lax.iota(jnp.int32, 16)
    vregs = [idxv[pl.ds(16 * i, 16)] for i in range(NVR)]
    h = [jnp.zeros((16,), jnp.int32) for _ in range(4)]
    for e in range(E):
        tot = jnp.zeros((16,), jnp.int32)
        for i in range(NVR):
            tot = tot + plsc.all_reduce_population_count(vregs[i] == e)
        k = e >> 4
        h[k] = h[k] + jnp.where(lane == (e & 15), tot, 0)
    for k in range(4):
        histv[pl.ds(16 * k, 16)] = h[k]
    pltpu.sync_copy(histv, hists_hbm.at[pl.ds(w * E, E)])


def _route_body(idx_hbm, w_hbm, x_hbm, hists_hbm,
                rft_hbm, wp_hbm, buf_hbm,
                idxv, wv, histsv, rowsv, scatv, wpv, xv, sem):
    w = _wid()
    xload = pltpu.async_copy(x_hbm.at[pl.ds(w * CH, CH)], xv, sem)
    pltpu.sync_copy(idx_hbm.at[pl.ds(w * CH, CH)], idxv)
    pltpu.sync_copy(w_hbm.at[pl.ds(w * CH, CH)], wv)
    pltpu.sync_copy(hists_hbm, histsv)
    lane = lax.iota(jnp.int32, 16)
    base = [jnp.zeros((16,), jnp.int32) for _ in range(4)]
    cnt = [jnp.zeros((16,), jnp.int32) for _ in range(4)]
    for wp_ in range(NW):
        pred = jnp.full((16,), wp_, jnp.int32) < w
        for k in range(4):
            hk = histsv[pl.ds(wp_ * E + 16 * k, 16)]
            cnt[k] = cnt[k] + hk
            base[k] = base[k] + jnp.where(pred, hk, 0)
    vregs = [idxv[pl.ds(16 * i, 16)] for i in range(NVR)]
    for i in range(NVR):
        v = vregs[i]
        rank = jnp.zeros((16,), jnp.int32)
        for j in range(i):
            u = vregs[j]
            for l in range(16):
                ul = jnp.take_along_axis(u, jnp.full((16,), l, jnp.int32), axis=0)
                rank = rank + (v == ul).astype(jnp.int32)
        for d in range(1, 16):
            rolled = jnp.take_along_axis(v, jnp.maximum(lane - d, 0), axis=0)
            rank = rank + ((rolled == v) & (lane >= d)).astype(jnp.int32)
        slot = _lookup64(base, v) + rank
        c_e = _lookup64(cnt, v)
        validv = slot < CAP
        row = v * CAP + jnp.minimum(slot, CAP - 1)
        scat = jnp.where(validv, row, TRASH + lane)
        dead = (slot == CAP - 1) & (c_e > CAP)
        wp = wv[pl.ds(16 * i, 16)] * (validv & ~dead).astype(jnp.float32)
        rowsv[pl.ds(16 * i, 16)] = row
        scatv[pl.ds(16 * i, 16)] = scat
        wpv[pl.ds(16 * i, 16)] = wp
    pltpu.sync_copy(rowsv, rft_hbm.at[pl.ds(w * CH, CH)])
    pltpu.sync_copy(wpv, wp_hbm.at[pl.ds(w * CH, CH)])
    xload.wait()
    pltpu.async_copy(xv, buf_hbm.at[scatv], sem).wait()


def _gather_body(rft_hbm, eo_hbm, routed_hbm, idxv, rowsb, sem):
    w = _wid()
    pltpu.sync_copy(rft_hbm.at[pl.ds(w * CH, CH)], idxv)
    pltpu.async_copy(eo_hbm.at[idxv], rowsb, sem).wait()
    pltpu.sync_copy(rowsb, routed_hbm.at[pl.ds(w * CH, CH)])


def _dotT(a, b):
    return lax.dot_general(a, b, (((1,), (1,)), ((), ())),
                           preferred_element_type=jnp.float32)


def _expert_ffn_body(buf_ref, gw_ref, uw_ref, dw_ref, out_ref):
    for e2 in range(EPB):
        x = buf_ref[pl.ds(e2 * CAP, CAP), :].astype(jnp.bfloat16)
        g = _dotT(x, gw_ref[e2].astype(jnp.bfloat16))
        u = _dotT(x, uw_ref[e2].astype(jnp.bfloat16))
        act = (g * jax.nn.sigmoid(g)) * u
        out_ref[pl.ds(e2 * CAP, CAP), :] = _dotT(
            act.astype(jnp.bfloat16), dw_ref[e2].astype(jnp.bfloat16))


def _shared_ffn_body(x_ref, gw_ref, uw_ref, dw_ref, out_ref):
    x = x_ref[...].astype(jnp.bfloat16)
    g = _dotT(x, gw_ref[...].astype(jnp.bfloat16))
    u = _dotT(x, uw_ref[...].astype(jnp.bfloat16))
    act = (g * jax.nn.sigmoid(g)) * u
    o = _dotT(act.astype(jnp.bfloat16), dw_ref[...].astype(jnp.bfloat16))
    out_ref[...] = o.astype(jnp.bfloat16)


def _combine_body(sh_ref, r_ref, wp_ref, out_ref):
    wcol = jnp.reshape(wp_ref[...], (sh_ref.shape[0], 1))
    out_ref[...] = sh_ref[...].astype(jnp.float32) + r_ref[...] * wcol


def kernel(x, topk_indices, topk_weights, gate_w, up_w, down_w,
           shared_gate_w, shared_up_w, shared_down_w):
    xf = x.reshape(T, H)
    idx = topk_indices.reshape(T).astype(jnp.int32)
    wts = topk_weights.reshape(T)

    mesh = plsc.VectorSubcoreMesh(core_axis_name="c", subcore_axis_name="s")

    hist_k = pl.kernel(
        _hist_body,
        out_type=jax.ShapeDtypeStruct((NW * E,), jnp.int32),
        mesh=mesh,
        compiler_params=pltpu.CompilerParams(needs_layout_passes=False),
        scratch_types=[pltpu.VMEM((CH,), jnp.int32),
                       pltpu.VMEM((E,), jnp.int32)],
    )
    hists = hist_k(idx)

    route_k = pl.kernel(
        _route_body,
        out_type=[jax.ShapeDtypeStruct((T,), jnp.int32),
                  jax.ShapeDtypeStruct((T,), jnp.float32),
                  jax.ShapeDtypeStruct((BUF_ROWS, H), jnp.float32)],
        mesh=mesh,
        compiler_params=pltpu.CompilerParams(needs_layout_passes=False),
        scratch_types=[pltpu.VMEM((CH,), jnp.int32),
                       pltpu.VMEM((CH,), jnp.float32),
                       pltpu.VMEM((NW * E,), jnp.int32),
                       pltpu.VMEM((CH,), jnp.int32),
                       pltpu.VMEM((CH,), jnp.int32),
                       pltpu.VMEM((CH,), jnp.float32),
                       pltpu.VMEM((CH, H), jnp.float32),
                       pltpu.SemaphoreType.DMA],
    )
    rft, wprime, buf = route_k(idx, wts, xf, hists)

    ST = 2048
    shared = pl.pallas_call(
        _shared_ffn_body,
        grid=(T // ST,),
        in_specs=[
            pl.BlockSpec((ST, H), lambda i: (i, 0)),
            pl.BlockSpec((H, H), lambda i: (0, 0)),
            pl.BlockSpec((H, H), lambda i: (0, 0)),
            pl.BlockSpec((H, H), lambda i: (0, 0)),
        ],
        out_specs=pl.BlockSpec((ST, H), lambda i: (i, 0)),
        out_shape=jax.ShapeDtypeStruct((T, H), jnp.bfloat16),
    )(xf, shared_gate_w, shared_up_w, shared_down_w)

    shared, buf = lax.optimization_barrier((shared, buf))

    expert_out = pl.pallas_call(
        _expert_ffn_body,
        grid=(E // EPB,),
        in_specs=[
            pl.BlockSpec((EPB * CAP, H), lambda e: (e, 0)),
            pl.BlockSpec((EPB, H, H), lambda e: (e, 0, 0)),
            pl.BlockSpec((EPB, H, H), lambda e: (e, 0, 0)),
            pl.BlockSpec((EPB, H, H), lambda e: (e, 0, 0)),
        ],
        out_specs=pl.BlockSpec((EPB * CAP, H), lambda e: (e, 0)),
        out_shape=jax.ShapeDtypeStruct((E * CAP, H), jnp.float32),
        compiler_params=pltpu.CompilerParams(
            vmem_limit_bytes=100 * 1024 * 1024),
    )(buf, gate_w, up_w, down_w)

    gather_k = pl.kernel(
        _gather_body,
        out_type=jax.ShapeDtypeStruct((T, H), jnp.float32),
        mesh=mesh,
        compiler_params=pltpu.CompilerParams(needs_layout_passes=False),
        scratch_types=[pltpu.VMEM((CH,), jnp.int32),
                       pltpu.VMEM((CH, H), jnp.float32),
                       pltpu.SemaphoreType.DMA],
    )
    routed = gather_k(rft, expert_out)

    TT = 1024
    out = pl.pallas_call(
        _combine_body,
        grid=(T // TT,),
        in_specs=[
            pl.BlockSpec((TT, H), lambda i: (i, 0)),
            pl.BlockSpec((TT, H), lambda i: (i, 0)),
            pl.BlockSpec((TT,), lambda i: (i,)),
        ],
        out_specs=pl.BlockSpec((TT, H), lambda i: (i, 0)),
        out_shape=jax.ShapeDtypeStruct((T, H), jnp.float32),
    )(shared, routed, wprime)

    return out.reshape(G, S, H)


# speedup vs baseline: 1.0265x; 1.0015x over previous
import jax
import jax.numpy as jnp
from jax import lax
from jax.experimental import pallas as pl
from jax.experimental.pallas import tpu as pltpu
from jax.experimental.pallas import tpu_sc as plsc

G, S, H = 2, 2048, 768
E, K = 64, 1
CAP = 128
T = G * S
NW = 32
CH = T // NW
NVR = CH // 16
BUF_ROWS = E * CAP + CH
TRASH = E * CAP
EPB = 2


def _wid():
    return lax.axis_index("s") * 2 + lax.axis_index("c")


def _lookup64(tab4, v):
    sel = lax.shift_right_logical(v, 4)
    off = jnp.bitwise_and(v, 15)
    r = jnp.zeros((16,), jnp.int32)
    for k in range(4):
        g = jnp.take_along_axis(tab4[k], off, axis=0)
        r = jnp.where(sel == k, g, r)
    return r


def _hist_body(idx_hbm, hists_hbm, idxv, histv):
    w = _wid()
    pltpu.sync_copy(idx_hbm.at[pl.ds(w * CH, CH)], idxv)
    lane = lax.iota(jnp.int32, 16)
    vregs = [idxv[pl.ds(16 * i, 16)] for i in range(NVR)]
    h = [jnp.zeros((16,), jnp.int32) for _ in range(4)]
    for e in range(E):
        tot = jnp.zeros((16,), jnp.int32)
        for i in range(NVR):
            tot = tot + plsc.all_reduce_population_count(vregs[i] == e)
        k = e >> 4
        h[k] = h[k] + jnp.where(lane == (e & 15), tot, 0)
    for k in range(4):
        histv[pl.ds(16 * k, 16)] = h[k]
    pltpu.sync_copy(histv, hists_hbm.at[pl.ds(w * E, E)])


def _route_body(idx_hbm, w_hbm, x_hbm, hists_hbm,
                rft_hbm, wp_hbm, buf_hbm,
                idxv, wv, histsv, rowsv, scatv, wpv, xv, sem):
    w = _wid()
    xload = pltpu.async_copy(x_hbm.at[pl.ds(w * CH, CH)], xv, sem)
    pltpu.sync_copy(idx_hbm.at[pl.ds(w * CH, CH)], idxv)
    pltpu.sync_copy(w_hbm.at[pl.ds(w * CH, CH)], wv)
    pltpu.sync_copy(hists_hbm, histsv)
    lane = lax.iota(jnp.int32, 16)
    base = [jnp.zeros((16,), jnp.int32) for _ in range(4)]
    cnt = [jnp.zeros((16,), jnp.int32) for _ in range(4)]
    for wp_ in range(NW):
        pred = jnp.full((16,), wp_, jnp.int32) < w
        for k in range(4):
            hk = histsv[pl.ds(wp_ * E + 16 * k, 16)]
            cnt[k] = cnt[k] + hk
            base[k] = base[k] + jnp.where(pred, hk, 0)
    vregs = [idxv[pl.ds(16 * i, 16)] for i in range(NVR)]
    for i in range(NVR):
        v = vregs[i]
        rank = jnp.zeros((16,), jnp.int32)
        for j in range(i):
            u = vregs[j]
            for l in range(16):
                ul = jnp.take_along_axis(u, jnp.full((16,), l, jnp.int32), axis=0)
                rank = rank + (v == ul).astype(jnp.int32)
        for d in range(1, 16):
            rolled = jnp.take_along_axis(v, jnp.maximum(lane - d, 0), axis=0)
            rank = rank + ((rolled == v) & (lane >= d)).astype(jnp.int32)
        slot = _lookup64(base, v) + rank
        c_e = _lookup64(cnt, v)
        validv = slot < CAP
        row = v * CAP + jnp.minimum(slot, CAP - 1)
        scat = jnp.where(validv, row, TRASH + lane)
        dead = (slot == CAP - 1) & (c_e > CAP)
        wp = wv[pl.ds(16 * i, 16)] * (validv & ~dead).astype(jnp.float32)
        rowsv[pl.ds(16 * i, 16)] = row
        scatv[pl.ds(16 * i, 16)] = scat
        wpv[pl.ds(16 * i, 16)] = wp
    pltpu.sync_copy(rowsv, rft_hbm.at[pl.ds(w * CH, CH)])
    pltpu.sync_copy(wpv, wp_hbm.at[pl.ds(w * CH, CH)])
    xload.wait()
    pltpu.async_copy(xv, buf_hbm.at[scatv], sem).wait()


def _gather_body(rft_hbm, eo_hbm, routed_hbm, idxv, rowsb, sem):
    w = _wid()
    pltpu.sync_copy(rft_hbm.at[pl.ds(w * CH, CH)], idxv)
    pltpu.async_copy(eo_hbm.at[idxv], rowsb, sem).wait()
    pltpu.sync_copy(rowsb, routed_hbm.at[pl.ds(w * CH, CH)])


def _dotT(a, b):
    return lax.dot_general(a, b, (((1,), (1,)), ((), ())),
                           preferred_element_type=jnp.float32)


def _expert_ffn_body(buf_ref, gw_ref, uw_ref, dw_ref, out_ref):
    for e2 in range(EPB):
        x = buf_ref[pl.ds(e2 * CAP, CAP), :].astype(jnp.bfloat16)
        g = _dotT(x, gw_ref[e2].astype(jnp.bfloat16))
        u = _dotT(x, uw_ref[e2].astype(jnp.bfloat16))
        act = (g * jax.nn.sigmoid(g)) * u
        out_ref[pl.ds(e2 * CAP, CAP), :] = _dotT(
            act.astype(jnp.bfloat16), dw_ref[e2].astype(jnp.bfloat16))


def _shared_ffn_body(x_ref, gw_ref, uw_ref, dw_ref, out_ref):
    x = x_ref[...].astype(jnp.bfloat16)
    g = _dotT(x, gw_ref[...].astype(jnp.bfloat16))
    u = _dotT(x, uw_ref[...].astype(jnp.bfloat16))
    act = (g * jax.nn.sigmoid(g)) * u
    o = _dotT(act.astype(jnp.bfloat16), dw_ref[...].astype(jnp.bfloat16))
    out_ref[...] = o.astype(jnp.bfloat16)


def _combine_body(sh_ref, r_ref, wp_ref, out_ref):
    wcol = jnp.reshape(wp_ref[...], (sh_ref.shape[0], 1))
    out_ref[...] = sh_ref[...].astype(jnp.float32) + r_ref[...] * wcol


def kernel(x, topk_indices, topk_weights, gate_w, up_w, down_w,
           shared_gate_w, shared_up_w, shared_down_w):
    xf = x.reshape(T, H)
    idx = topk_indices.reshape(T).astype(jnp.int32)
    wts = topk_weights.reshape(T)

    mesh = plsc.VectorSubcoreMesh(core_axis_name="c", subcore_axis_name="s")

    hist_k = pl.kernel(
        _hist_body,
        out_type=jax.ShapeDtypeStruct((NW * E,), jnp.int32),
        mesh=mesh,
        compiler_params=pltpu.CompilerParams(needs_layout_passes=False),
        scratch_types=[pltpu.VMEM((CH,), jnp.int32),
                       pltpu.VMEM((E,), jnp.int32)],
    )
    hists = hist_k(idx)

    route_k = pl.kernel(
        _route_body,
        out_type=[jax.ShapeDtypeStruct((T,), jnp.int32),
                  jax.ShapeDtypeStruct((T,), jnp.float32),
                  jax.ShapeDtypeStruct((BUF_ROWS, H), jnp.float32)],
        mesh=mesh,
        compiler_params=pltpu.CompilerParams(needs_layout_passes=False),
        scratch_types=[pltpu.VMEM((CH,), jnp.int32),
                       pltpu.VMEM((CH,), jnp.float32),
                       pltpu.VMEM((NW * E,), jnp.int32),
                       pltpu.VMEM((CH,), jnp.int32),
                       pltpu.VMEM((CH,), jnp.int32),
                       pltpu.VMEM((CH,), jnp.float32),
                       pltpu.VMEM((CH, H), jnp.float32),
                       pltpu.SemaphoreType.DMA],
    )
    rft, wprime, buf = route_k(idx, wts, xf, hists)

    ST = 2048
    shared = pl.pallas_call(
        _shared_ffn_body,
        grid=(T // ST,),
        in_specs=[
            pl.BlockSpec((ST, H), lambda i: (i, 0)),
            pl.BlockSpec((H, H), lambda i: (0, 0)),
            pl.BlockSpec((H, H), lambda i: (0, 0)),
            pl.BlockSpec((H, H), lambda i: (0, 0)),
        ],
        out_specs=pl.BlockSpec((ST, H), lambda i: (i, 0)),
        out_shape=jax.ShapeDtypeStruct((T, H), jnp.bfloat16),
    )(xf, shared_gate_w, shared_up_w, shared_down_w)

    shared, buf = lax.optimization_barrier((shared, buf))

    expert_out = pl.pallas_call(
        _expert_ffn_body,
        grid=(E // EPB,),
        in_specs=[
            pl.BlockSpec((EPB * CAP, H), lambda e: (e, 0)),
            pl.BlockSpec((EPB, H, H), lambda e: (e, 0, 0)),
            pl.BlockSpec((EPB, H, H), lambda e: (e, 0, 0)),
            pl.BlockSpec((EPB, H, H), lambda e: (e, 0, 0)),
        ],
        out_specs=pl.BlockSpec((EPB * CAP, H), lambda e: (e, 0)),
        out_shape=jax.ShapeDtypeStruct((E * CAP, H), jnp.float32),
        compiler_params=pltpu.CompilerParams(
            vmem_limit_bytes=100 * 1024 * 1024),
    )(buf, gate_w, up_w, down_w)

    gather_k = pl.kernel(
        _gather_body,
        out_type=jax.ShapeDtypeStruct((T, H), jnp.float32),
        mesh=mesh,
        compiler_params=pltpu.CompilerParams(needs_layout_passes=False),
        scratch_types=[pltpu.VMEM((CH,), jnp.int32),
                       pltpu.VMEM((CH, H), jnp.float32),
                       pltpu.SemaphoreType.DMA],
    )
    routed = gather_k(rft, expert_out)

    TT = 1024
    out = pl.pallas_call(
        _combine_body,
        grid=(T // TT,),
        in_specs=[
            pl.BlockSpec((TT, H), lambda i: (i, 0)),
            pl.BlockSpec((TT, H), lambda i: (i, 0)),
            pl.BlockSpec((TT,), lambda i: (i,)),
        ],
        out_specs=pl.BlockSpec((TT, H), lambda i: (i, 0)),
        out_shape=jax.ShapeDtypeStruct((T, H), jnp.float32),
    )(shared, routed, wprime)

    return out.reshape(G, S, H)
